# baseline (speedup 1.0000x reference)
import functools

import jax
import jax.numpy as jnp
from jax import lax
from jax.experimental import pallas as pl
from jax.experimental.pallas import tpu as pltpu
from jax.experimental.pallas import tpu_sc as plsc

_NC = 2
_NS = 16
_NW = _NC * _NS

_B = 4096
_S = 200
_D = 32
_Q = 8
_U = _B // _Q
_UPW = _S * _Q // _NW
_BT = _U // 128


def _gather_body(idxT_hbm, table_hbm, z_hbm, idx_v, rows_v, z_v, gsem, wsem):
    wid = lax.axis_index("s") * _NC + lax.axis_index("c")

    lane = lax.iota(jnp.int32, 16)

    def unit_body(i, carry):
        u = wid * _UPW + i
        l = u // _Q
        q = lax.rem(u, _Q)

        pltpu.sync_copy(idxT_hbm.at[l, pl.ds(q * _U, _U)], idx_v)
        pltpu.async_copy(table_hbm.at[idx_v], rows_v, gsem).wait()

        def tbody(r, c2):
            dt = r // (_BT * 8)
            b8 = lax.rem(r // 8, _BT)
            ds = lax.rem(r, 8)
            col = jnp.full((16,), dt * 8 + ds, jnp.int32)
            base = b8 * 128
            for bl0 in range(0, 128, 16):
                v = plsc.load_gather(rows_v, [base + bl0 + lane, col])
                z_v[dt, b8, ds, pl.ds(bl0, 16)] = v
            return c2

        lax.fori_loop(0, 4 * _BT * 8, tbody, 0)

        ws = []
        for dt in range(4):
            ws.append(
                pltpu.async_copy(
                    z_v.at[dt], z_hbm.at[l, dt, pl.ds(q * _BT, _BT)], wsem
                )
            )
        for w in ws:
            w.wait()
        return carry

    lax.fori_loop(0, _UPW, unit_body, 0)


@functools.partial(
    pl.kernel,
    out_type=jax.ShapeDtypeStruct((_S, 4, _B // 128, 8, 128), jnp.float32),
    mesh=plsc.VectorSubcoreMesh(core_axis_name="c", subcore_axis_name="s"),
    scratch_types=[
        pltpu.VMEM((_U,), jnp.int32),
        pltpu.VMEM((_U, _D), jnp.float32),
        pltpu.VMEM((4, _BT, 8, 128), jnp.float32),
        pltpu.SemaphoreType.DMA,
        pltpu.SemaphoreType.DMA,
    ],
    compiler_params=pltpu.CompilerParams(
        use_tc_tiling_on_sc=False, needs_layout_passes=False
    ),
)
def _gather_call(idxT_hbm, table_hbm, z_hbm, idx_v, rows_v, z_v, gsem, wsem):
    _gather_body(idxT_hbm, table_hbm, z_hbm, idx_v, rows_v, z_v, gsem, wsem)


@jax.jit
def kernel(token_ids, embedding_table):
    z = _gather_call(token_ids.T.astype(jnp.int32), embedding_table)
    return z.transpose(2, 4, 0, 1, 3).reshape(_B, _S, _D)


# speedup vs baseline: 1.0649x; 1.0649x over previous
import functools

import jax
import jax.numpy as jnp
from jax import lax
from jax.experimental import pallas as pl
from jax.experimental.pallas import tpu as pltpu
from jax.experimental.pallas import tpu_sc as plsc

_NC = 2
_NS = 16
_NW = _NC * _NS

_B = 4096
_S = 200
_D = 32
_Q = 8
_U = _B // _Q
_UPW = _S * _Q // _NW
_BT = _U // 128
_PITCH = 33


def _gather_body(idxT_hbm, table_hbm, z_hbm, idx_v, rows_v, pad_v, z_v, gsem, wsem):
    wid = lax.axis_index("s") * _NC + lax.axis_index("c")

    lane = lax.iota(jnp.int32, 16)
    u0 = wid * _UPW

    def stage_and_fire(i, slot):
        l = (u0 + i) // _Q
        q = lax.rem(u0 + i, _Q)
        pltpu.sync_copy(idxT_hbm.at[l, pl.ds(q * _U, _U)], idx_v.at[slot])
        pltpu.async_copy(table_hbm.at[idx_v.at[slot]], rows_v.at[slot], gsem)

    stage_and_fire(0, 0)

    def unit_body(i, carry):
        s = lax.rem(i, 2)
        l = (u0 + i) // _Q
        q = lax.rem(u0 + i, _Q)
        rows = rows_v.at[s]
        z = z_v.at[s]

        @pl.when(i + 1 < _UPW)
        def _():
            stage_and_fire(i + 1, 1 - s)

        pltpu.make_async_copy(table_hbm.at[pl.ds(0, _U)], rows, gsem).wait()

        @pl.when(i >= 2)
        def _():
            for dt in range(4):
                pltpu.make_async_copy(
                    z_hbm.at[0, dt, pl.ds(0, _BT)], z_v.at[s, dt], wsem
                ).wait()

        for c in range(_BT):
            def stage_body(t, c2):
                for h in range(2):
                    pad_v[t, pl.ds(16 * h, 16)] = rows[c * 128 + t, pl.ds(16 * h, 16)]
                return c2

            lax.fori_loop(0, 128, stage_body, 0)
            for d in range(_D):
                col = jnp.full((16,), d, jnp.int32)
                for bl0 in range(0, 128, 16):
                    v = plsc.load_gather(pad_v, [bl0 + lane, col])
                    z[d // 8, c, d % 8, pl.ds(bl0, 16)] = v

        for dt in range(4):
            pltpu.async_copy(
                z.at[dt], z_hbm.at[l, dt, pl.ds(q * _BT, _BT)], wsem
            )
        return carry

    lax.fori_loop(0, _UPW, unit_body, 0)

    for s in range(2):
        for dt in range(4):
            pltpu.make_async_copy(
                z_hbm.at[0, dt, pl.ds(0, _BT)], z_v.at[s, dt], wsem
            ).wait()


@functools.partial(
    pl.kernel,
    out_type=jax.ShapeDtypeStruct((_S, 4, _B // 128, 8, 128), jnp.float32),
    mesh=plsc.VectorSubcoreMesh(core_axis_name="c", subcore_axis_name="s"),
    scratch_types=[
        pltpu.VMEM((2, _U), jnp.int32),
        pltpu.VMEM((2, _U, _D), jnp.float32),
        pltpu.VMEM((128, _PITCH), jnp.float32),
        pltpu.VMEM((2, 4, _BT, 8, 128), jnp.float32),
        pltpu.SemaphoreType.DMA,
        pltpu.SemaphoreType.DMA,
    ],
    compiler_params=pltpu.CompilerParams(
        use_tc_tiling_on_sc=False, needs_layout_passes=False
    ),
)
def _gather_call(idxT_hbm, table_hbm, z_hbm, idx_v, rows_v, pad_v, z_v, gsem, wsem):
    _gather_body(idxT_hbm, table_hbm, z_hbm, idx_v, rows_v, pad_v, z_v, gsem, wsem)


@jax.jit
def kernel(token_ids, embedding_table):
    z = _gather_call(token_ids.T.astype(jnp.int32), embedding_table)
    return z.transpose(2, 4, 0, 1, 3).reshape(_B, _S, _D)


# speedup vs baseline: 1.0880x; 1.0218x over previous
import functools

import jax
import jax.numpy as jnp
from jax import lax
from jax.experimental import pallas as pl
from jax.experimental.pallas import tpu as pltpu
from jax.experimental.pallas import tpu_sc as plsc

_NC = 2
_NS = 16
_NW = _NC * _NS

_B = 4096
_S = 200
_D = 32
_Q = 8
_U = _B // _Q
_UPW = _S * _Q // _NW
_BT = _U // 128
_PITCH = 33


def _gather_body(idxT_hbm, table_hbm, z_hbm, idx_v, rows_v, pad_v, z_v, gsem, wsem):
    wid = lax.axis_index("s") * _NC + lax.axis_index("c")

    lane = lax.iota(jnp.int32, 16)
    u0 = wid * _UPW

    def stage_and_fire(i, slot):
        l = (u0 + i) // _Q
        q = lax.rem(u0 + i, _Q)
        pltpu.sync_copy(idxT_hbm.at[l, pl.ds(q * _U, _U)], idx_v.at[slot])
        pltpu.async_copy(table_hbm.at[idx_v.at[slot]], rows_v.at[slot], gsem)

    stage_and_fire(0, 0)

    def unit_body(i, carry):
        s = lax.rem(i, 2)
        l = (u0 + i) // _Q
        q = lax.rem(u0 + i, _Q)
        rows = rows_v.at[s]
        z = z_v.at[s]

        @pl.when(i + 1 < _UPW)
        def _():
            stage_and_fire(i + 1, 1 - s)

        pltpu.make_async_copy(table_hbm.at[pl.ds(0, _U)], rows, gsem).wait()

        @pl.when(i >= 2)
        def _():
            for dt in range(4):
                pltpu.make_async_copy(
                    z_hbm.at[0, dt, pl.ds(0, _BT)], z_v.at[s, dt], wsem
                ).wait()

        for c in range(_BT):
            def stage_body(t8, c2):
                for t1 in range(8):
                    for h in range(2):
                        pad_v[t8 * 8 + t1, pl.ds(16 * h, 16)] = rows[
                            c * 128 + t8 * 8 + t1, pl.ds(16 * h, 16)
                        ]
                return c2

            lax.fori_loop(0, 16, stage_body, 0)
            for d in range(_D):
                col = jnp.full((16,), d, jnp.int32)
                for bl0 in range(0, 128, 16):
                    v = plsc.load_gather(pad_v, [bl0 + lane, col])
                    z[d // 8, c, d % 8, pl.ds(bl0, 16)] = v

        for dt in range(4):
            pltpu.async_copy(
                z.at[dt], z_hbm.at[l, dt, pl.ds(q * _BT, _BT)], wsem
            )
        return carry

    lax.fori_loop(0, _UPW, unit_body, 0)

    for s in range(2):
        for dt in range(4):
            pltpu.make_async_copy(
                z_hbm.at[0, dt, pl.ds(0, _BT)], z_v.at[s, dt], wsem
            ).wait()


@functools.partial(
    pl.kernel,
    out_type=jax.ShapeDtypeStruct((_S, 4, _B // 128, 8, 128), jnp.float32),
    mesh=plsc.VectorSubcoreMesh(core_axis_name="c", subcore_axis_name="s"),
    scratch_types=[
        pltpu.VMEM((2, _U), jnp.int32),
        pltpu.VMEM((2, _U, _D), jnp.float32),
        pltpu.VMEM((128, _PITCH), jnp.float32),
        pltpu.VMEM((2, 4, _BT, 8, 128), jnp.float32),
        pltpu.SemaphoreType.DMA,
        pltpu.SemaphoreType.DMA,
    ],
    compiler_params=pltpu.CompilerParams(
        use_tc_tiling_on_sc=False, needs_layout_passes=False
    ),
)
def _gather_call(idxT_hbm, table_hbm, z_hbm, idx_v, rows_v, pad_v, z_v, gsem, wsem):
    _gather_body(idxT_hbm, table_hbm, z_hbm, idx_v, rows_v, pad_v, z_v, gsem, wsem)


@jax.jit
def kernel(token_ids, embedding_table):
    z = _gather_call(token_ids.T.astype(jnp.int32), embedding_table)
    return z.transpose(2, 4, 0, 1, 3).reshape(_B, _S, _D)


# speedup vs baseline: 1.2600x; 1.1581x over previous
import functools

import jax
import jax.numpy as jnp
from jax import lax
from jax.experimental import pallas as pl
from jax.experimental.pallas import tpu as pltpu
from jax.experimental.pallas import tpu_sc as plsc

_NC = 2
_NS = 16
_NW = _NC * _NS

_B = 4096
_S = 200
_D = 32
_R = _B * _S
_RPW = _R // _NW
_CH = 1024
_NCH = _RPW // _CH
_K = 1
_BLK = _K * _CH
_NBLK = _NCH // _K


def _gather_body(idx_hbm, table_hbm, out_hbm, idx_v, rows_v, sem, wsem):
    wid = lax.axis_index("s") * _NC + lax.axis_index("c")
    ibase = wid * _NCH
    obase = wid * _RPW

    pltpu.sync_copy(idx_hbm.at[pl.ds(ibase, _NCH)], idx_v)

    def blk_body(blk, carry):
        slot = lax.rem(blk, 2)
        rows = rows_v.at[slot]

        @pl.when(blk >= 2)
        def _():
            pltpu.make_async_copy(out_hbm.at[pl.ds(obase, _BLK)], rows, wsem).wait()

        waits = []
        for k in range(_K):
            waits.append(
                pltpu.async_copy(
                    table_hbm.at[idx_v.at[blk * _K + k]],
                    rows.at[pl.ds(k * _CH, _CH)],
                    sem,
                )
            )
        for w in waits:
            w.wait()
        pltpu.async_copy(rows, out_hbm.at[pl.ds(obase + blk * _BLK, _BLK)], wsem)
        return carry

    lax.fori_loop(0, _NBLK, blk_body, 0)

    for slot in range(2):
        pltpu.make_async_copy(
            out_hbm.at[pl.ds(obase, _BLK)], rows_v.at[slot], wsem
        ).wait()


@functools.partial(
    pl.kernel,
    out_type=jax.ShapeDtypeStruct((_R, _D), jnp.float32),
    mesh=plsc.VectorSubcoreMesh(core_axis_name="c", subcore_axis_name="s"),
    scratch_types=[
        pltpu.VMEM((_NCH, _CH), jnp.int32),
        pltpu.VMEM((2, _BLK, _D), jnp.float32),
        pltpu.SemaphoreType.DMA,
        pltpu.SemaphoreType.DMA,
    ],
    compiler_params=pltpu.CompilerParams(use_tc_tiling_on_sc=False),
)
def _gather_call(idx_hbm, table_hbm, out_hbm, idx_v, rows_v, sem, wsem):
    _gather_body(idx_hbm, table_hbm, out_hbm, idx_v, rows_v, sem, wsem)


@jax.jit
def kernel(token_ids, embedding_table):
    idx = token_ids.reshape(_R // _CH, _CH).astype(jnp.int32)
    out = _gather_call(idx, embedding_table)
    return out.reshape(_B, _S, _D)
